# baseline (speedup 1.0000x reference)
import functools
import math

import jax
import jax.numpy as jnp
from jax import lax
from jax.experimental import pallas as pl
from jax.experimental.pallas import tpu as pltpu
from jax.experimental.pallas import tpu_sc as plsc

VOCAB_ = 1000000
DIM_ = 64
SCALE_ = math.sqrt(DIM_)

NC = 2
NS = 16
NW = NC * NS
LANES = 16


def _make_sc_embed(B: int, C: int):
  assert B % NW == 0
  b_per_w = B // NW
  assert b_per_w % C == 0
  nchunks = b_per_w // C
  mesh = plsc.VectorSubcoreMesh(core_axis_name="c", subcore_axis_name="s")

  @functools.partial(
      pl.kernel,
      out_type=jax.ShapeDtypeStruct((B, DIM_), jnp.float32),
      mesh=mesh,
      scratch_types=[
          pltpu.VMEM((C,), jnp.int32),
          pltpu.VMEM((C, DIM_), jnp.float32),
          pltpu.SemaphoreType.DMA,
      ],
      compiler_params=pltpu.CompilerParams(use_tc_tiling_on_sc=False),
  )
  def embed(x_hbm, table_hbm, out_hbm, idx_v, rows_v, sem):
    wid = lax.axis_index("s") * NC + lax.axis_index("c")
    wbase = wid * b_per_w

    @pl.loop(0, nchunks)
    def _chunk(k):
      base = wbase + k * C
      pltpu.sync_copy(x_hbm.at[pl.ds(base, C)], idx_v)
      pltpu.async_copy(table_hbm.at[idx_v], rows_v, sem).wait()

      @pl.loop(0, C)
      def _row(r):
        for c in range(DIM_ // LANES):
          sl = pl.ds(c * LANES, LANES)
          rows_v[r, sl] = rows_v[r, sl] * SCALE_

      pltpu.sync_copy(rows_v, out_hbm.at[pl.ds(base, C)])

  return embed


def kernel(x, table):
  B = x.shape[0] * x.shape[1]
  flat = x.reshape(B).astype(jnp.int32)
  out = _make_sc_embed(B, 512)(flat, table)
  return out.reshape(x.shape[0], x.shape[1], DIM_)


# speedup vs baseline: 1.1363x; 1.1363x over previous
import functools
import math

import jax
import jax.numpy as jnp
from jax import lax
from jax.experimental import pallas as pl
from jax.experimental.pallas import tpu as pltpu
from jax.experimental.pallas import tpu_sc as plsc

VOCAB_ = 1000000
DIM_ = 64
SCALE_ = math.sqrt(DIM_)

NC = 2
NS = 16
NW = NC * NS
LANES = 16


def _make_sc_embed(B: int, C: int):
  assert B % NW == 0
  b_per_w = B // NW
  assert b_per_w % C == 0
  nchunks = b_per_w // C
  assert nchunks >= 4 and nchunks % 2 == 0
  mesh = plsc.VectorSubcoreMesh(core_axis_name="c", subcore_axis_name="s")

  @functools.partial(
      pl.kernel,
      out_type=jax.ShapeDtypeStruct((B, DIM_), jnp.float32),
      mesh=mesh,
      scratch_types=[
          pltpu.VMEM((b_per_w,), jnp.int32),
          pltpu.VMEM((C, DIM_), jnp.float32),
          pltpu.VMEM((C, DIM_), jnp.float32),
          pltpu.SemaphoreType.DMA,
          pltpu.SemaphoreType.DMA,
          pltpu.SemaphoreType.DMA,
          pltpu.SemaphoreType.DMA,
      ],
      compiler_params=pltpu.CompilerParams(use_tc_tiling_on_sc=False),
  )
  def embed(x_hbm, table_hbm, out_hbm, idx_all, rows0, rows1, sg0, sg1,
            ss0, ss1):
    wid = lax.axis_index("s") * NC + lax.axis_index("c")
    wbase = wid * b_per_w

    def g_start(k, rows, sem):
      pltpu.async_copy(table_hbm.at[idx_all.at[pl.ds(k * C, C)]], rows, sem)

    def g_wait(rows, sem):
      pltpu.make_async_copy(
          table_hbm.at[idx_all.at[pl.ds(0, C)]], rows, sem).wait()

    def s_start(k, rows, sem):
      pltpu.async_copy(rows, out_hbm.at[pl.ds(wbase + k * C, C)], sem)

    def s_wait(rows, sem):
      pltpu.make_async_copy(rows, out_hbm.at[pl.ds(wbase, C)], sem).wait()

    def scale(rows):
      @plsc.parallel_loop(0, C, 1, unroll=8)
      def _(r):
        for c in range(DIM_ // LANES):
          sl = pl.ds(c * LANES, LANES)
          rows[r, sl] = rows[r, sl] * SCALE_

    pltpu.sync_copy(x_hbm.at[pl.ds(wbase, b_per_w)], idx_all)

    g_start(0, rows0, sg0)
    g_wait(rows0, sg0)
    g_start(1, rows1, sg1)
    scale(rows0)
    s_start(0, rows0, ss0)

    @pl.loop(1, nchunks - 1, step=2)
    def _(k0):
      g_wait(rows1, sg1)
      s_wait(rows0, ss0)
      g_start(k0 + 1, rows0, sg0)
      scale(rows1)
      s_start(k0, rows1, ss1)

      g_wait(rows0, sg0)
      s_wait(rows1, ss1)
      g_start(k0 + 2, rows1, sg1)
      scale(rows0)
      s_start(k0 + 1, rows0, ss0)

    g_wait(rows1, sg1)
    s_wait(rows0, ss0)
    scale(rows1)
    s_start(nchunks - 1, rows1, ss1)
    s_wait(rows1, ss1)

  return embed


def kernel(x, table):
  B = x.shape[0] * x.shape[1]
  flat = x.reshape(B).astype(jnp.int32)
  out = _make_sc_embed(B, 640)(flat, table)
  return out.reshape(x.shape[0], x.shape[1], DIM_)


# speedup vs baseline: 1.1662x; 1.0263x over previous
import functools
import math

import jax
import jax.numpy as jnp
from jax import lax
from jax.experimental import pallas as pl
from jax.experimental.pallas import tpu as pltpu
from jax.experimental.pallas import tpu_sc as plsc

VOCAB_ = 1000000
DIM_ = 64
SCALE_ = math.sqrt(DIM_)

NC = 2
NS = 16
NW = NC * NS
LANES = 16


def _make_sc_embed(B: int, C: int):
  assert B % NW == 0
  b_per_w = B // NW
  assert b_per_w % C == 0
  nchunks = b_per_w // C
  assert nchunks >= 4 and nchunks % 2 == 0
  mesh = plsc.VectorSubcoreMesh(core_axis_name="c", subcore_axis_name="s")

  @functools.partial(
      pl.kernel,
      out_type=jax.ShapeDtypeStruct((B, DIM_), jnp.float32),
      mesh=mesh,
      scratch_types=[
          pltpu.VMEM((b_per_w,), jnp.int32),
          pltpu.VMEM((C, DIM_), jnp.float32),
          pltpu.VMEM((C, DIM_), jnp.float32),
          pltpu.SemaphoreType.DMA,
          pltpu.SemaphoreType.DMA,
          pltpu.SemaphoreType.DMA,
          pltpu.SemaphoreType.DMA,
      ],
      compiler_params=pltpu.CompilerParams(use_tc_tiling_on_sc=False),
  )
  def embed(x_hbm, table_hbm, out_hbm, idx_all, rows0, rows1, sg0, sg1,
            ss0, ss1):
    wid = lax.axis_index("s") * NC + lax.axis_index("c")
    wbase = wid * b_per_w

    def g_start(k, rows, sem):
      pltpu.async_copy(table_hbm.at[idx_all.at[pl.ds(k * C, C)]], rows, sem)

    def g_wait(rows, sem):
      pltpu.make_async_copy(
          table_hbm.at[idx_all.at[pl.ds(0, C)]], rows, sem).wait()

    def s_start(k, rows, sem):
      pltpu.async_copy(rows, out_hbm.at[pl.ds(wbase + k * C, C)], sem)

    def s_wait(rows, sem):
      pltpu.make_async_copy(rows, out_hbm.at[pl.ds(wbase, C)], sem).wait()

    def scale(rows):
      @plsc.parallel_loop(0, C, 1, unroll=8)
      def _(r):
        for c in range(DIM_ // LANES):
          sl = pl.ds(c * LANES, LANES)
          rows[r, sl] = rows[r, sl] * SCALE_

    pltpu.sync_copy(x_hbm.at[pl.ds(wbase, b_per_w)], idx_all)

    g_start(0, rows0, sg0)
    g_wait(rows0, sg0)
    g_start(1, rows1, sg1)
    scale(rows0)
    s_start(0, rows0, ss0)

    @pl.loop(1, nchunks - 1, step=2)
    def _(k0):
      g_wait(rows1, sg1)
      s_wait(rows0, ss0)
      g_start(k0 + 1, rows0, sg0)
      scale(rows1)
      s_start(k0, rows1, ss1)

      g_wait(rows0, sg0)
      s_wait(rows1, ss1)
      g_start(k0 + 2, rows1, sg1)
      scale(rows0)
      s_start(k0 + 1, rows0, ss0)

    g_wait(rows1, sg1)
    s_wait(rows0, ss0)
    scale(rows1)
    s_start(nchunks - 1, rows1, ss1)
    s_wait(rows1, ss1)

  return embed


def kernel(x, table):
  nb, nj = x.shape
  B = nb * nj
  flat = x.T.reshape(B)
  out = _make_sc_embed(B, 640)(flat, table)
  return out.reshape(nj, nb, DIM_).transpose(1, 0, 2)
